# baseline (speedup 1.0000x reference)
import functools

import jax
import jax.numpy as jnp
from jax import lax
from jax.experimental import pallas as pl
from jax.experimental.pallas import tpu as pltpu

V = 1_000_000
E = 64
CTX = 200


def _tc_gather(idx_ref, tab_hbm, out_ref, rows, sem):
    for i in range(CTX):
        pltpu.make_async_copy(
            tab_hbm.at[pl.ds(idx_ref[i], 1)], rows.at[pl.ds(i, 1)], sem
        ).start()
    for i in range(CTX):
        pltpu.make_async_copy(
            tab_hbm.at[pl.ds(idx_ref[i], 1)], rows.at[pl.ds(i, 1)], sem
        ).wait()
    out_ref[...] = jnp.sum(rows[...], axis=0, keepdims=True)


_gather = pl.pallas_call(
    _tc_gather,
    grid_spec=pltpu.PrefetchScalarGridSpec(
        num_scalar_prefetch=1,
        grid=(1,),
        in_specs=[pl.BlockSpec(memory_space=pltpu.MemorySpace.HBM)],
        out_specs=pl.BlockSpec((1, E), lambda i, *_: (0, 0)),
        scratch_shapes=[
            pltpu.VMEM((CTX, E), jnp.float32),
            pltpu.SemaphoreType.DMA,
        ],
    ),
    out_shape=jax.ShapeDtypeStruct((1, E), jnp.float32),
)


BLKV = 8_192
NBV = (V + BLKV - 1) // BLKV


def _tc_matvec(emb_ref, w_ref, b_ref, out_ref):
    res = lax.dot_general(emb_ref[...], w_ref[...], (((1,), (1,)), ((), ())),
                          preferred_element_type=jnp.float32)
    out_ref[...] = res[0] + b_ref[...]


_matvec = pl.pallas_call(
    _tc_matvec,
    grid=(NBV,),
    in_specs=[
        pl.BlockSpec((1, E), lambda i: (0, 0)),
        pl.BlockSpec((BLKV, E), lambda i: (i, 0)),
        pl.BlockSpec((BLKV,), lambda i: (i,)),
    ],
    out_specs=pl.BlockSpec((BLKV,), lambda i: (i,)),
    out_shape=jax.ShapeDtypeStruct((V,), jnp.float32),
    compiler_params=pltpu.CompilerParams(
        dimension_semantics=("parallel",)
    ),
)


def kernel(inputs, emb_table, W, b):
    emb = _gather(inputs, emb_table)
    return _matvec(emb, W, b)


# speedup vs baseline: 1.0299x; 1.0299x over previous
import functools

import jax
import jax.numpy as jnp
from jax import lax
from jax.experimental import pallas as pl
from jax.experimental.pallas import tpu as pltpu

V = 1_000_000
E = 64
CTX = 200


def _tc_gather(idx_ref, tab_hbm, out_ref, rows, sem):
    for i in range(CTX):
        pltpu.make_async_copy(
            tab_hbm.at[pl.ds(idx_ref[i], 1)], rows.at[pl.ds(i, 1)], sem
        ).start()
    for i in range(CTX):
        pltpu.make_async_copy(
            tab_hbm.at[pl.ds(idx_ref[i], 1)], rows.at[pl.ds(i, 1)], sem
        ).wait()
    out_ref[...] = jnp.sum(rows[...], axis=0, keepdims=True)


_gather = pl.pallas_call(
    _tc_gather,
    grid_spec=pltpu.PrefetchScalarGridSpec(
        num_scalar_prefetch=1,
        grid=(1,),
        in_specs=[pl.BlockSpec(memory_space=pltpu.MemorySpace.HBM)],
        out_specs=pl.BlockSpec((1, E), lambda i, *_: (0, 0)),
        scratch_shapes=[
            pltpu.VMEM((CTX, E), jnp.float32),
            pltpu.SemaphoreType.DMA,
        ],
    ),
    out_shape=jax.ShapeDtypeStruct((1, E), jnp.float32),
)


BLKV = 32_768
NBV = (V + BLKV - 1) // BLKV


def _tc_matvec(emb_ref, w_ref, b_ref, out_ref):
    res = lax.dot_general(emb_ref[...], w_ref[...], (((1,), (1,)), ((), ())),
                          preferred_element_type=jnp.float32)
    out_ref[...] = res[0] + b_ref[...]


_matvec = pl.pallas_call(
    _tc_matvec,
    grid=(NBV,),
    in_specs=[
        pl.BlockSpec((1, E), lambda i: (0, 0)),
        pl.BlockSpec((BLKV, E), lambda i: (i, 0)),
        pl.BlockSpec((BLKV,), lambda i: (i,)),
    ],
    out_specs=pl.BlockSpec((BLKV,), lambda i: (i,)),
    out_shape=jax.ShapeDtypeStruct((V,), jnp.float32),
)


def kernel(inputs, emb_table, W, b):
    emb = _gather(inputs, emb_table)
    return _matvec(emb, W, b)
